# baseline (speedup 1.0000x reference)
import jax
import jax.numpy as jnp
from jax import lax
from jax.experimental import pallas as pl
from jax.experimental.pallas import tpu as pltpu
from jax.experimental.pallas import tpu_sc as plsc

NC = 2
NS = 16
NW = NC * NS

V = 1000000
NB = 16384
NS_TOK = 26
D = 64
LANES = 128

V_MAIN = V - LANES
N_RBLK = (V_MAIN + LANES - 1) // LANES
RBLK_PER_W = (N_RBLK + NW - 1) // NW

N_UNITS = NS_TOK * (NB // LANES)
UNITS_PER_W = N_UNITS // NW


def _widen_body(tableT_hbm, tail_hbm, wide_hbm, slab_v, blk_v, lane_v):
    wid = lax.axis_index("s") * NC + lax.axis_index("c")
    lane_v[...] = lax.iota(jnp.int32, 16)

    def transpose_cols(n_cols, src_ref):
        def row(r, carry):
            rvec = jnp.full((16,), r, jnp.int32)
            for g in range(D // 16):
                blk_v[r, pl.ds(g * 16, 16)] = plsc.load_gather(
                    src_ref, [lane_v[...] + g * 16, rvec]
                )
            return carry

        lax.fori_loop(0, n_cols, row, 0, unroll=2)

    def step(k, carry):
        rblk = wid + k * NW

        @pl.when(rblk < N_RBLK)
        def _():
            r0 = pl.multiple_of(rblk * LANES, LANES)
            pltpu.sync_copy(tableT_hbm.at[:, pl.ds(r0, LANES)], slab_v)
            transpose_cols(LANES, slab_v)
            pltpu.sync_copy(blk_v, wide_hbm.at[pl.ds(r0, LANES)])

        return carry

    lax.fori_loop(0, RBLK_PER_W, step, 0)

    @pl.when(wid == 0)
    def _():
        pltpu.sync_copy(tail_hbm, slab_v)
        transpose_cols(LANES, slab_v)
        pltpu.sync_copy(blk_v, wide_hbm.at[pl.ds(V_MAIN, LANES)])


def _gather_body(idxT_hbm, wide_hbm, out2_hbm, idx_v, rows_v, blk_v, lane_v, sem):
    wid = lax.axis_index("s") * NC + lax.axis_index("c")
    lane_v[...] = lax.iota(jnp.int32, 16)
    nb_blk = NB // LANES

    def unit(u, carry):
        uu = wid * UNITS_PER_W + u
        s = uu // nb_blk
        b0 = (uu % nb_blk) * LANES
        pltpu.sync_copy(idxT_hbm.at[s, pl.ds(b0, LANES)], idx_v)
        pltpu.async_copy(wide_hbm.at[idx_v], rows_v, sem).wait()

        def col(c, carry2):
            cvec = jnp.full((16,), c, jnp.int32)
            for g in range(LANES // 16):
                blk_v[c, pl.ds(g * 16, 16)] = plsc.load_gather(
                    rows_v, [lane_v[...] + g * 16, cvec]
                )
            return carry2

        lax.fori_loop(0, D, col, 0, unroll=2)
        pltpu.sync_copy(blk_v, out2_hbm.at[s, :, pl.ds(b0, LANES)])
        return carry

    lax.fori_loop(0, UNITS_PER_W, unit, 0)


def kernel(indices, embedding_matrix):
    tableT = embedding_matrix.T
    tail = lax.slice(tableT, (0, V - LANES), (D, V))
    idxT = indices.T.astype(jnp.int32)
    mesh = plsc.VectorSubcoreMesh(core_axis_name="c", subcore_axis_name="s")
    widen = pl.kernel(
        _widen_body,
        out_type=jax.ShapeDtypeStruct((V, LANES), jnp.float32),
        mesh=mesh,
        scratch_types=[
            pltpu.VMEM((D, LANES), jnp.float32),
            pltpu.VMEM((LANES, LANES), jnp.float32),
            pltpu.VMEM((16,), jnp.int32),
        ],
        compiler_params=pltpu.CompilerParams(use_tc_tiling_on_sc=True, needs_layout_passes=False),
    )
    gather = pl.kernel(
        _gather_body,
        out_type=jax.ShapeDtypeStruct((NS_TOK, D, NB), jnp.float32),
        mesh=mesh,
        scratch_types=[
            pltpu.VMEM((LANES,), jnp.int32),
            pltpu.VMEM((LANES, LANES), jnp.float32),
            pltpu.VMEM((D, LANES), jnp.float32),
            pltpu.VMEM((16,), jnp.int32),
            pltpu.SemaphoreType.DMA,
        ],
        compiler_params=pltpu.CompilerParams(use_tc_tiling_on_sc=True, needs_layout_passes=False),
    )
    wide = widen(tableT, tail)
    out2 = gather(idxT, wide)
    return jnp.transpose(out2, (2, 0, 1))


# speedup vs baseline: 1.6532x; 1.6532x over previous
import jax
import jax.numpy as jnp
from jax import lax
from jax.experimental import pallas as pl
from jax.experimental.pallas import tpu as pltpu
from jax.experimental.pallas import tpu_sc as plsc

NC = 2
NS = 16
NW = NC * NS

V = 1000000
NB = 16384
NS_TOK = 26
D = 64
LANES = 128

V_MAIN = V - LANES
N_RBLK = (V_MAIN + LANES - 1) // LANES
RBLK_PER_W = (N_RBLK + NW - 1) // NW

N_UNITS = NS_TOK * (NB // LANES)
UNITS_PER_W = N_UNITS // NW


def _widen_tc_body(tT_ref, wide_ref):
    wide_ref[:, 0:D] = tT_ref[...].T


def _gather_body(idxT_hbm, wide_hbm, out2_hbm, idx_v, rows_v, blk_v, lane_v, sem):
    wid = lax.axis_index("s") * NC + lax.axis_index("c")
    lane_v[...] = lax.iota(jnp.int32, 16)
    nb_blk = NB // LANES

    def unit(u, carry):
        uu = wid * UNITS_PER_W + u
        s = uu // nb_blk
        b0 = (uu % nb_blk) * LANES
        pltpu.sync_copy(idxT_hbm.at[s, pl.ds(b0, LANES)], idx_v)
        pltpu.async_copy(wide_hbm.at[idx_v], rows_v, sem).wait()

        def col(c, carry2):
            cvec = jnp.full((16,), c, jnp.int32)
            for g in range(LANES // 16):
                blk_v[c, pl.ds(g * 16, 16)] = plsc.load_gather(
                    rows_v, [lane_v[...] + g * 16, cvec]
                )
            return carry2

        lax.fori_loop(0, D, col, 0, unroll=8)
        pltpu.sync_copy(blk_v, out2_hbm.at[s, :, pl.ds(b0, LANES)])
        return carry

    lax.fori_loop(0, UNITS_PER_W, unit, 0)


def kernel(indices, embedding_matrix):
    tableT = embedding_matrix.T
    idxT = indices.T.astype(jnp.int32)
    BL = 512
    n_blk = (V + BL - 1) // BL
    widen = pl.pallas_call(
        _widen_tc_body,
        grid=(n_blk,),
        in_specs=[pl.BlockSpec((D, BL), lambda i: (0, i))],
        out_specs=pl.BlockSpec((BL, LANES), lambda i: (i, 0)),
        out_shape=jax.ShapeDtypeStruct((V, LANES), jnp.float32),
        compiler_params=pltpu.CompilerParams(
            dimension_semantics=("arbitrary",)
        ),
    )
    mesh = plsc.VectorSubcoreMesh(core_axis_name="c", subcore_axis_name="s")
    gather = pl.kernel(
        _gather_body,
        out_type=jax.ShapeDtypeStruct((NS_TOK, D, NB), jnp.float32),
        mesh=mesh,
        scratch_types=[
            pltpu.VMEM((LANES,), jnp.int32),
            pltpu.VMEM((LANES, LANES), jnp.float32),
            pltpu.VMEM((D, LANES), jnp.float32),
            pltpu.VMEM((16,), jnp.int32),
            pltpu.SemaphoreType.DMA,
        ],
        compiler_params=pltpu.CompilerParams(use_tc_tiling_on_sc=True, needs_layout_passes=False),
    )
    wide = widen(tableT)
    out2 = gather(idxT, wide)
    return jnp.transpose(out2, (2, 0, 1))


# speedup vs baseline: 5.4160x; 3.2761x over previous
import jax
import jax.numpy as jnp
from jax import lax
from jax.experimental import pallas as pl
from jax.experimental.pallas import tpu as pltpu
from jax.experimental.pallas import tpu_sc as plsc

NC = 2
NS = 16
NW = NC * NS

V = 1000000
NB = 16384
NS_TOK = 26
D = 64
LANES = 128

N_RBLK = (V - LANES) // LANES + 1
RBLK_PER_W = (N_RBLK + NW - 1) // NW
TAIL0 = V - LANES

B = NB * NS_TOK
B_PER_W = B // NW
N_UNITS_W = B_PER_W // LANES
NB_BLK = NB // LANES


def _skew_transpose(src_v, dst_v, src_rows, n_cols, iv):
    n_mc = n_cols // 16

    def blk16(m, carry):
        rvec = iv + (m // n_mc) * 16
        cbase = (m % n_mc) * 16
        for j in range(16):
            cvec = cbase + lax.rem(iv + j, 16)
            vals = plsc.load_gather(src_v, [rvec, cvec])
            plsc.store_scatter(dst_v, [cvec, rvec], vals)
        return carry

    lax.fori_loop(0, (src_rows // 16) * n_mc, blk16, 0, unroll=2)


def _widen_body(tableT_hbm, tail_hbm, wide_hbm, slab0, slab1, blk0, blk1,
                iota_v, ss0, ss1, sw0, sw1):
    wid = lax.axis_index("s") * NC + lax.axis_index("c")
    iota_v[...] = lax.iota(jnp.int32, 16)
    iv = iota_v[...]
    slabs = (slab0, slab1)
    blks = (blk0, blk1)
    sss = (ss0, ss1)
    sws = (sw0, sw1)

    def r0_of(u):
        return pl.multiple_of((wid + u * NW) * LANES, LANES)

    def slab_copy(u, par):
        return pltpu.make_async_copy(
            tableT_hbm.at[:, pl.ds(r0_of(u), LANES)], slabs[par], sss[par]
        )

    def wide_copy(u, par):
        return pltpu.make_async_copy(
            blks[par], wide_hbm.at[pl.ds(r0_of(u), LANES)], sws[par]
        )

    def in_range(u):
        return wid + u * NW < N_RBLK

    for par in (0, 1):
        @pl.when(in_range(par))
        def _():
            slab_copy(par, par).start()

    def phase(k2, par):
        u = k2 * 2 + par

        @pl.when(in_range(u))
        def _():
            slab_copy(u, par).wait()

            @pl.when(k2 >= 1)
            def _():
                wide_copy(u - 2, par).wait()

            _skew_transpose(slabs[par], blks[par], D, LANES, iv)
            wide_copy(u, par).start()

            @pl.when(in_range(u + 2))
            def _():
                slab_copy(u + 2, par).start()

    def pair(k2, carry):
        phase(k2, 0)
        phase(k2, 1)
        return carry

    lax.fori_loop(0, (RBLK_PER_W + 1) // 2, pair, 0)

    @pl.when(in_range(RBLK_PER_W - 2))
    def _():
        wide_copy(RBLK_PER_W - 2, (RBLK_PER_W - 2) % 2).wait()

    @pl.when(in_range(RBLK_PER_W - 1))
    def _():
        wide_copy(RBLK_PER_W - 1, (RBLK_PER_W - 1) % 2).wait()

    @pl.when(wid == 4)
    def _():
        pltpu.sync_copy(tail_hbm, slab0)
        _skew_transpose(slab0, blk0, D, LANES, iv)
        pltpu.sync_copy(blk0, wide_hbm.at[pl.ds(TAIL0, LANES)])


def _gather_body(idxf_hbm, wide_hbm, out2_hbm, idx_all, rows0, rows1,
                 blk0, blk1, iota_v, sg0, sg1, so0, so1):
    wid = lax.axis_index("s") * NC + lax.axis_index("c")
    iota_v[...] = lax.iota(jnp.int32, 16)
    iv = iota_v[...]
    rows = (rows0, rows1)
    blks = (blk0, blk1)
    sgs = (sg0, sg1)
    sos = (so0, so1)
    uu0 = wid * N_UNITS_W

    pltpu.sync_copy(
        idxf_hbm.at[pl.ds(pl.multiple_of(wid * B_PER_W, 8), B_PER_W)],
        idx_all,
    )

    def gather_copy(u, par):
        idx_ref = idx_all.at[pl.ds(pl.multiple_of(u * LANES, 8), LANES)]
        return pltpu.make_async_copy(
            wide_hbm.at[idx_ref], rows[par], sgs[par]
        )

    def out_copy(u, par):
        s = (uu0 + u) // NB_BLK
        b0 = pl.multiple_of(((uu0 + u) % NB_BLK) * LANES, LANES)
        return pltpu.make_async_copy(
            blks[par], out2_hbm.at[s, :, pl.ds(b0, LANES)], sos[par]
        )

    gather_copy(0, 0).start()
    gather_copy(1, 1).start()

    def phase(k2, par):
        u = k2 * 2 + par
        gather_copy(u, par).wait()

        @pl.when(k2 >= 1)
        def _():
            out_copy(u - 2, par).wait()

        _skew_transpose(rows[par], blks[par], LANES, D, iv)
        out_copy(u, par).start()

        @pl.when(k2 < N_UNITS_W // 2 - 1)
        def _():
            gather_copy(u + 2, par).start()

    def pair(k2, carry):
        phase(k2, 0)
        phase(k2, 1)
        return carry

    lax.fori_loop(0, N_UNITS_W // 2, pair, 0)
    out_copy(N_UNITS_W - 2, 0).wait()
    out_copy(N_UNITS_W - 1, 1).wait()


def kernel(indices, embedding_matrix):
    tableT = embedding_matrix.T
    tail = lax.slice(tableT, (0, TAIL0), (D, V))
    idxT_flat = indices.T.astype(jnp.int32).reshape(-1)
    mesh = plsc.VectorSubcoreMesh(core_axis_name="c", subcore_axis_name="s")
    sc_params = pltpu.CompilerParams(
        use_tc_tiling_on_sc=True, needs_layout_passes=False
    )
    widen = pl.kernel(
        _widen_body,
        out_type=jax.ShapeDtypeStruct((V, LANES), jnp.float32),
        mesh=mesh,
        scratch_types=[
            pltpu.VMEM((D, LANES), jnp.float32),
            pltpu.VMEM((D, LANES), jnp.float32),
            pltpu.VMEM((LANES, LANES), jnp.float32),
            pltpu.VMEM((LANES, LANES), jnp.float32),
            pltpu.VMEM((16,), jnp.int32),
            pltpu.SemaphoreType.DMA,
            pltpu.SemaphoreType.DMA,
            pltpu.SemaphoreType.DMA,
            pltpu.SemaphoreType.DMA,
        ],
        compiler_params=sc_params,
    )
    gather = pl.kernel(
        _gather_body,
        out_type=jax.ShapeDtypeStruct((NS_TOK, D, NB), jnp.float32),
        mesh=mesh,
        scratch_types=[
            pltpu.VMEM((B_PER_W,), jnp.int32),
            pltpu.VMEM((LANES, LANES), jnp.float32),
            pltpu.VMEM((LANES, LANES), jnp.float32),
            pltpu.VMEM((D, LANES), jnp.float32),
            pltpu.VMEM((D, LANES), jnp.float32),
            pltpu.VMEM((16,), jnp.int32),
            pltpu.SemaphoreType.DMA,
            pltpu.SemaphoreType.DMA,
            pltpu.SemaphoreType.DMA,
            pltpu.SemaphoreType.DMA,
        ],
        compiler_params=sc_params,
    )
    wide = widen(tableT, tail)
    out2 = gather(idxT_flat, wide)
    return jnp.transpose(out2, (2, 0, 1))
